# Initial kernel scaffold; baseline (speedup 1.0000x reference)
import jax
import jax.numpy as jnp
from jax.experimental import pallas as pl


def kernel(x):
    raise NotImplementedError("write your pallas kernel here")


# speedup vs baseline: 4.2450x; 4.2450x over previous
import functools

import jax
import jax.numpy as jnp
from jax.experimental import pallas as pl

_BONUS = 10.0


def _wp(v):
    return jnp.concatenate([v[..., 1:], v[..., -1:]], axis=-1)


def _wm(v):
    return jnp.concatenate([v[..., :1], v[..., :-1]], axis=-1)


def _dp(v):
    return jnp.concatenate([v[:, 1:], v[:, -1:]], axis=1)


def _dm(v):
    return jnp.concatenate([v[:, :1], v[:, :-1]], axis=1)


def _stencil_kernel(ht, h_total, x_ref, coords_ref, y_ref):
    i = pl.program_id(0)
    rs = i * ht
    c = x_ref[:, :, pl.ds(rs, ht), :]
    top = x_ref[:, :, pl.ds(jnp.maximum(rs - 1, 0), 1), :]
    bot = x_ref[:, :, pl.ds(jnp.minimum(rs + ht, h_total - 1), 1), :]
    ext = jnp.concatenate([top, c, bot], axis=2)

    wp = _wp(ext)
    wm = _wm(ext)
    dpe = _dp(ext)
    dme = _dm(ext)
    hp = ext[:, :, 2:]
    hm = ext[:, :, :-2]

    wpc = wp[:, :, 1:-1]
    wmc = wm[:, :, 1:-1]
    dpc = dpe[:, :, 1:-1]
    dmc = dme[:, :, 1:-1]

    b0 = 0.5 * (wpc - wmc)
    b1 = 0.5 * (hp - hm)
    b2 = 0.5 * (dpc - dmc)

    h00 = wmc - 2.0 * c + wpc
    h11 = hm - 2.0 * c + hp
    h22 = dmc - 2.0 * c + dpc
    dxy = wm[:, :, :-2] - wp[:, :, :-2] - wm[:, :, 2:] + wp[:, :, 2:]
    dys = -dme[:, :, :-2] + dme[:, :, 2:] + dpe[:, :, :-2] - dpe[:, :, 2:]
    dxs = -_wm(dmc) + _wp(dmc) + _wm(dpc) - _wp(dpc)
    h01 = 0.25 * dxy
    h12 = 0.25 * dys
    h02 = 0.25 * dxs

    det = (h00 * (h11 * h22 - h12 * h12)
           - h01 * (h01 * h22 - h12 * h02)
           + h02 * (h01 * h12 - h11 * h02))
    valid = jnp.isfinite(det) & (jnp.abs(det) > 0.0)
    recip = 1.0 / jnp.where(valid, det, 1.0)

    a00 = h11 * h22 - h12 * h12
    a01 = h02 * h12 - h01 * h22
    a02 = h01 * h12 - h02 * h11
    a11 = h00 * h22 - h02 * h02
    a12 = h01 * h02 - h00 * h12
    a22 = h00 * h11 - h01 * h01
    x0 = (a00 * b0 + a01 * b1 + a02 * b2) * recip
    x1 = (a01 * b0 + a11 * b1 + a12 * b2) * recip
    x2 = (a02 * b0 + a12 * b1 + a22 * b2) * recip

    mw = jnp.maximum(jnp.maximum(wm, ext), wp)
    mh = jnp.maximum(jnp.maximum(mw[:, :, :-2], mw[:, :, 1:-1]), mw[:, :, 2:])
    md = jnp.maximum(jnp.maximum(_dm(mh), mh), _dp(mh))
    nms = (c == md) & valid

    dx0 = -jnp.where(nms, x0, 0.0)
    dx1 = -jnp.where(nms, x1, 0.0)
    dx2 = -jnp.where(nms, x2, 0.0)
    max_abs = jnp.maximum(jnp.maximum(jnp.abs(dx0), jnp.abs(dx1)), jnp.abs(dx2))
    keep = max_abs <= 0.7
    dx0 = jnp.where(keep, dx0, 0.0)
    dx1 = jnp.where(keep, dx1, 0.0)
    dx2 = jnp.where(keep, dx2, 0.0)
    dy = 0.5 * (b0 * dx0 + b1 * dx1 + b2 * dx2)

    y_ref[...] = c + dy + _BONUS * nms.astype(c.dtype)

    didx = jax.lax.broadcasted_iota(jnp.int32, c.shape, 1).astype(c.dtype)
    hidx = (rs + jax.lax.broadcasted_iota(jnp.int32, c.shape, 2)).astype(c.dtype)
    widx = jax.lax.broadcasted_iota(jnp.int32, c.shape, 3).astype(c.dtype)
    coords_ref[:, 0] = didx + dx2
    coords_ref[:, 1] = widx + dx0
    coords_ref[:, 2] = hidx + dx1


def kernel(x):
    B, C, D, H, W = x.shape
    xs = x.reshape(B * C, D, H, W)
    HT = 64
    grid = (H // HT,)
    coords, y = pl.pallas_call(
        functools.partial(_stencil_kernel, HT, H),
        grid=grid,
        in_specs=[
            pl.BlockSpec((B * C, D, H, W), lambda i: (0, 0, 0, 0)),
        ],
        out_specs=[
            pl.BlockSpec((B * C, 3, D, HT, W), lambda i: (0, 0, 0, i, 0)),
            pl.BlockSpec((B * C, D, HT, W), lambda i: (0, 0, i, 0)),
        ],
        out_shape=[
            jax.ShapeDtypeStruct((B * C, 3, D, H, W), x.dtype),
            jax.ShapeDtypeStruct((B * C, D, H, W), x.dtype),
        ],
    )(xs)
    return coords.reshape(B, C, 3, D, H, W), y.reshape(B, C, D, H, W)


# speedup vs baseline: 6.6479x; 1.5660x over previous
import functools

import jax
import jax.numpy as jnp
from jax.experimental import pallas as pl

_BONUS = 10.0


def _wp(v):
    return jnp.concatenate([v[..., 1:], v[..., -1:]], axis=-1)


def _wm(v):
    return jnp.concatenate([v[..., :1], v[..., :-1]], axis=-1)


def _dext(v):
    return jnp.concatenate([v[:, :1], v, v[:, -1:]], axis=1)


def _stencil_kernel(ht, h_total, x_ref, coords_ref, y_ref):
    i = pl.program_id(0)
    rs = i * ht
    c = x_ref[:, :, pl.ds(rs, ht), :]
    trow = x_ref[:, :, pl.ds(jnp.maximum(rs - 1, 0), 1), :]
    brow = x_ref[:, :, pl.ds(jnp.minimum(rs + ht, h_total - 1), 1), :]

    Cx = _dext(c)
    top = _dext(trow)
    bot = _dext(brow)

    CHp = jnp.concatenate([Cx[:, :, 1:], bot], axis=2)
    CHm = jnp.concatenate([top, Cx[:, :, :-1]], axis=2)
    CWp = _wp(Cx)
    CWm = _wm(Cx)

    K = CWm - CWp
    L = CHp - CHm

    cc = Cx[:, 1:5]
    K4 = K[:, 1:5]
    ktop = _wm(trow) - _wp(trow)
    kbot = _wm(brow) - _wp(brow)
    KHm = jnp.concatenate([ktop, K4[:, :, :-1]], axis=2)
    KHp = jnp.concatenate([K4[:, :, 1:], kbot], axis=2)

    b0 = -0.5 * K4
    b1 = 0.5 * L[:, 1:5]
    b2 = 0.5 * (Cx[:, 2:6] - Cx[:, 0:4])

    two_c = cc + cc
    h00 = (CWm[:, 1:5] + CWp[:, 1:5]) - two_c
    h11 = (CHm[:, 1:5] + CHp[:, 1:5]) - two_c
    h22 = (Cx[:, 2:6] + Cx[:, 0:4]) - two_c
    h01 = 0.25 * (KHm - KHp)
    h12 = 0.25 * (L[:, 0:4] - L[:, 2:6])
    h02 = 0.25 * (K[:, 2:6] - K[:, 0:4])

    a00 = h11 * h22 - h12 * h12
    a01 = h02 * h12 - h01 * h22
    a02 = h01 * h12 - h02 * h11
    a11 = h00 * h22 - h02 * h02
    a12 = h01 * h02 - h00 * h12
    a22 = h00 * h11 - h01 * h01
    det = h00 * a00 + h01 * a01 + h02 * a02
    valid = jnp.abs(det) > 0.0
    recip = 1.0 / jnp.where(valid, det, 1.0)
    x0 = (a00 * b0 + a01 * b1 + a02 * b2) * recip
    x1 = (a01 * b0 + a11 * b1 + a12 * b2) * recip
    x2 = (a02 * b0 + a12 * b1 + a22 * b2) * recip

    m1 = jnp.maximum(jnp.maximum(CHm, Cx), CHp)
    mh = jnp.maximum(jnp.maximum(_wm(m1), m1), _wp(m1))
    md = jnp.maximum(jnp.maximum(mh[:, 0:4], mh[:, 1:5]), mh[:, 2:6])
    nms = (cc == md) & valid

    amax = jnp.maximum(jnp.maximum(jnp.abs(x0), jnp.abs(x1)), jnp.abs(x2))
    take = nms & (amax <= 0.7)
    dx0 = jnp.where(take, -x0, 0.0)
    dx1 = jnp.where(take, -x1, 0.0)
    dx2 = jnp.where(take, -x2, 0.0)
    dy = 0.5 * (b0 * dx0 + b1 * dx1 + b2 * dx2)

    y_ref[...] = cc + dy + _BONUS * nms.astype(cc.dtype)

    didx = jax.lax.broadcasted_iota(jnp.int32, cc.shape, 1).astype(cc.dtype)
    hidx = (rs + jax.lax.broadcasted_iota(jnp.int32, cc.shape, 2)).astype(cc.dtype)
    widx = jax.lax.broadcasted_iota(jnp.int32, cc.shape, 3).astype(cc.dtype)
    coords_ref[:, 0] = didx + dx2
    coords_ref[:, 1] = widx + dx0
    coords_ref[:, 2] = hidx + dx1


def kernel(x):
    B, C, D, H, W = x.shape
    xs = x.reshape(B * C, D, H, W)
    HT = 64
    grid = (H // HT,)
    coords, y = pl.pallas_call(
        functools.partial(_stencil_kernel, HT, H),
        grid=grid,
        in_specs=[
            pl.BlockSpec((B * C, D, H, W), lambda i: (0, 0, 0, 0)),
        ],
        out_specs=[
            pl.BlockSpec((B * C, 3, D, HT, W), lambda i: (0, 0, 0, i, 0)),
            pl.BlockSpec((B * C, D, HT, W), lambda i: (0, 0, i, 0)),
        ],
        out_shape=[
            jax.ShapeDtypeStruct((B * C, 3, D, H, W), x.dtype),
            jax.ShapeDtypeStruct((B * C, D, H, W), x.dtype),
        ],
    )(xs)
    return coords.reshape(B, C, 3, D, H, W), y.reshape(B, C, D, H, W)
